# baseline (speedup 1.0000x reference)
import functools

import jax
import jax.numpy as jnp
from jax.experimental import pallas as pl


_NEG = -1e30


def _body(x_p_ref, x_q_ref, w_p_ref, w_q_ref, b_p_ref, b_q_ref, oput_ref,
          size_ref, loss_ref, logp_ref, logq_ref):
    v = w_p_ref.shape[1]
    sz = size_ref[...]
    iota = jax.lax.broadcasted_iota(jnp.int32, (1, v), 1)
    mask = iota < sz
    oput = oput_ref[...]

    def one_side(x_ref, w_ref, b_ref):
        x = x_ref[...].astype(jnp.bfloat16)
        w = w_ref[...].astype(jnp.bfloat16)
        logits = jnp.dot(x, w, preferred_element_type=jnp.float32)
        logits = logits + b_ref[...]
        ml = jnp.where(mask, logits, jnp.float32(_NEG))
        m = jnp.max(ml, axis=1, keepdims=True)
        s = jnp.sum(jnp.exp(ml - m), axis=1, keepdims=True)
        lse = m + jnp.log(s)
        raw = jnp.sum(ml * oput, axis=1, keepdims=True)
        return raw - lse

    lp = one_side(x_p_ref, w_p_ref, b_p_ref)
    lq = one_side(x_q_ref, w_q_ref, b_q_ref)
    logp_ref[...] = lp
    logq_ref[...] = lq
    loss_ref[...] = -lp - lq


@functools.partial(jax.jit, static_argnames=())
def kernel(p_iput, q_iput, true_oput, oput_size, W_p, b_p, W_q, b_q):
    B, D = p_iput.shape
    V = W_p.shape[1]
    BB = 256 if B % 256 == 0 else B
    grid = (B // BB,)

    size2d = oput_size.reshape(B, 1).astype(jnp.int32)
    bp2d = b_p.reshape(1, V)
    bq2d = b_q.reshape(1, V)

    out_shapes = [jax.ShapeDtypeStruct((B, 1), jnp.float32)] * 3
    loss, log_p, log_q = pl.pallas_call(
        _body,
        grid=grid,
        in_specs=[
            pl.BlockSpec((BB, D), lambda i: (i, 0)),
            pl.BlockSpec((BB, D), lambda i: (i, 0)),
            pl.BlockSpec((D, V), lambda i: (0, 0)),
            pl.BlockSpec((D, V), lambda i: (0, 0)),
            pl.BlockSpec((1, V), lambda i: (0, 0)),
            pl.BlockSpec((1, V), lambda i: (0, 0)),
            pl.BlockSpec((BB, V), lambda i: (i, 0)),
            pl.BlockSpec((BB, 1), lambda i: (i, 0)),
        ],
        out_specs=[pl.BlockSpec((BB, 1), lambda i: (i, 0))] * 3,
        out_shape=out_shapes,
    )(p_iput, q_iput, W_p, W_q, bp2d, bq2d, true_oput, size2d)

    return (true_oput, loss[:, 0], log_p[:, 0], log_q[:, 0])


# speedup vs baseline: 1.0227x; 1.0227x over previous
import functools

import jax
import jax.numpy as jnp
from jax.experimental import pallas as pl


_NEG = -1e30


def _body(x_p_ref, x_q_ref, w_p_ref, w_q_ref, b_p_ref, b_q_ref, oput_ref,
          size_ref, loss_ref, logp_ref, logq_ref):
    v = w_p_ref.shape[1]
    sz = size_ref[...]
    iota = jax.lax.broadcasted_iota(jnp.int32, (1, v), 1)
    mask = iota < sz
    oput = oput_ref[...]

    def one_side(x_ref, w_ref, b_ref):
        x = x_ref[...].astype(jnp.bfloat16)
        logits = jnp.dot(x, w_ref[...], preferred_element_type=jnp.float32)
        logits = logits + b_ref[...]
        ml = jnp.where(mask, logits, jnp.float32(_NEG))
        m = jnp.max(ml, axis=1, keepdims=True)
        s = jnp.sum(jnp.exp(ml - m), axis=1, keepdims=True)
        lse = m + jnp.log(s)
        raw = jnp.sum(ml * oput, axis=1, keepdims=True)
        return raw - lse

    lp = one_side(x_p_ref, w_p_ref, b_p_ref)
    lq = one_side(x_q_ref, w_q_ref, b_q_ref)
    logp_ref[...] = lp
    logq_ref[...] = lq
    loss_ref[...] = -lp - lq


@functools.partial(jax.jit, static_argnames=())
def kernel(p_iput, q_iput, true_oput, oput_size, W_p, b_p, W_q, b_q):
    B, D = p_iput.shape
    V = W_p.shape[1]
    BB = 256 if B % 256 == 0 else B
    grid = (B // BB,)

    size2d = oput_size.reshape(B, 1).astype(jnp.int32)
    bp2d = b_p.reshape(1, V)
    bq2d = b_q.reshape(1, V)
    W_p = W_p.astype(jnp.bfloat16)
    W_q = W_q.astype(jnp.bfloat16)

    out_shapes = [jax.ShapeDtypeStruct((B, 1), jnp.float32)] * 3
    loss, log_p, log_q = pl.pallas_call(
        _body,
        grid=grid,
        in_specs=[
            pl.BlockSpec((BB, D), lambda i: (i, 0)),
            pl.BlockSpec((BB, D), lambda i: (i, 0)),
            pl.BlockSpec((D, V), lambda i: (0, 0)),
            pl.BlockSpec((D, V), lambda i: (0, 0)),
            pl.BlockSpec((1, V), lambda i: (0, 0)),
            pl.BlockSpec((1, V), lambda i: (0, 0)),
            pl.BlockSpec((BB, V), lambda i: (i, 0)),
            pl.BlockSpec((BB, 1), lambda i: (i, 0)),
        ],
        out_specs=[pl.BlockSpec((BB, 1), lambda i: (i, 0))] * 3,
        out_shape=out_shapes,
    )(p_iput, q_iput, W_p, W_q, bp2d, bq2d, true_oput, size2d)

    return (true_oput, loss[:, 0], log_p[:, 0], log_q[:, 0])
